# baseline (speedup 1.0000x reference)
import functools

import jax
import jax.numpy as jnp
from jax import lax
from jax.experimental import pallas as pl
from jax.experimental.pallas import tpu as pltpu
from jax.experimental.pallas import tpu_sc as plsc

_NC = 2
_NS = 16
_NW = _NC * _NS

_N = 10000
_NP = 10240
_E = 320000
_D = 128
_K = 512

_RB = 1024
_GRID = _NP // _RB

_EW = _E // _NW
_ECH = 200
_DCH = 2000
_ZROWS = 40

_SELU_ALPHA = 1.6732632423543772
_SELU_SCALE = 1.0507009873554805


def _mesh():
    return plsc.VectorSubcoreMesh(
        core_axis_name="c", subcore_axis_name="s",
        num_cores=_NC, num_subcores=_NS)


def _deg_body(send_hbm, recv_hbm, outs_hbm, outr_hbm,
              idx_v, ones_v, zb_v, hs_sh, hr_sh):
    cid = lax.axis_index("c")
    sid = lax.axis_index("s")
    w = cid * _NS + sid

    def fill_ones(i, carry):
        ones_v[pl.ds(i * 16, 16)] = jnp.full((16,), 1.0, jnp.float32)
        return carry
    lax.fori_loop(0, _DCH // 16, fill_ones, 0)

    def fill_zero(i, carry):
        zb_v[pl.ds(i * 16, 16)] = jnp.zeros((16,), jnp.float32)
        return carry
    lax.fori_loop(0, 640 // 16, fill_zero, 0)

    pltpu.sync_copy(zb_v, hs_sh.at[pl.ds(sid * 640, 640)])
    pltpu.sync_copy(zb_v, hr_sh.at[pl.ds(sid * 640, 640)])
    plsc.subcore_barrier()

    def step(i, carry):
        base = w * _EW + i * _DCH
        pltpu.sync_copy(send_hbm.at[pl.ds(base, _DCH)], idx_v)
        pltpu.sync_copy(ones_v, hs_sh.at[idx_v], add=True)
        pltpu.sync_copy(recv_hbm.at[pl.ds(base, _DCH)], idx_v)
        pltpu.sync_copy(ones_v, hr_sh.at[idx_v], add=True)
        return carry
    lax.fori_loop(0, _EW // _DCH, step, 0)
    plsc.subcore_barrier()

    pltpu.sync_copy(hs_sh.at[pl.ds(sid * 640, 640)],
                    outs_hbm.at[cid, pl.ds(sid * 640, 640)])
    pltpu.sync_copy(hr_sh.at[pl.ds(sid * 640, 640)],
                    outr_hbm.at[cid, pl.ds(sid * 640, 640)])


def _deg_call(send, recv):
    f = pl.kernel(
        _deg_body,
        out_type=[jax.ShapeDtypeStruct((_NC, _NP), jnp.float32),
                  jax.ShapeDtypeStruct((_NC, _NP), jnp.float32)],
        mesh=_mesh(),
        scratch_types=[
            pltpu.VMEM((_DCH,), jnp.int32),
            pltpu.VMEM((_DCH,), jnp.float32),
            pltpu.VMEM((640,), jnp.float32),
            pltpu.VMEM_SHARED((_NP,), jnp.float32),
            pltpu.VMEM_SHARED((_NP,), jnp.float32),
        ],
    )
    return f(send, recv)


_ACH = 128
_WCH = 80
_HNCH = 40
_EPAD = _NW * _WCH * _ACH


def _agg_body(tab1_hbm, tab2_hbm, send3d_hbm, recv3d_hbm, out1_hbm, out2_hbm,
              sidx_v, ridx_v, buf0, buf1, acc_sh, sem0, sem1):
    cid = lax.axis_index("c")
    sid = lax.axis_index("s")
    w = cid * _NS + sid

    for tab_hbm, out_hbm in ((tab1_hbm, out1_hbm), (tab2_hbm, out2_hbm)):
        def zrow(i, carry):
            for cix in range(_D // 16):
                buf0[i, pl.ds(cix * 16, 16)] = jnp.zeros((16,), jnp.float32)
            return carry
        lax.fori_loop(0, _ACH, zrow, 0)
        for j in range(640 // _ACH):
            pltpu.sync_copy(buf0,
                            acc_sh.at[pl.ds(sid * 640 + j * _ACH, _ACH)])
        plsc.subcore_barrier()

        def gstart(i, buf, sem):
            pltpu.make_async_copy(tab_hbm.at[sidx_v.at[i]], buf, sem).start()

        def gwait(i, buf, sem):
            pltpu.make_async_copy(tab_hbm.at[sidx_v.at[i]], buf, sem).wait()

        for h in range(2):
            pltpu.sync_copy(send3d_hbm.at[w, pl.ds(h * _HNCH, _HNCH)], sidx_v)
            pltpu.sync_copy(recv3d_hbm.at[w, pl.ds(h * _HNCH, _HNCH)], ridx_v)

            gstart(0, buf0, sem0)

            def step(i, carry):
                @pl.when(i % 2 == 0)
                def _():
                    gwait(i, buf0, sem0)

                    @pl.when(i + 1 < _HNCH)
                    def _():
                        gstart(i + 1, buf1, sem1)
                    pltpu.sync_copy(buf0, acc_sh.at[ridx_v.at[i]], add=True)

                @pl.when(i % 2 == 1)
                def _():
                    gwait(i, buf1, sem1)

                    @pl.when(i + 1 < _HNCH)
                    def _():
                        gstart(i + 1, buf0, sem0)
                    pltpu.sync_copy(buf1, acc_sh.at[ridx_v.at[i]], add=True)
                return carry
            lax.fori_loop(0, _HNCH, step, 0)
        plsc.subcore_barrier()

        for j in range(640 // _ACH):
            r0 = sid * 640 + j * _ACH
            pltpu.sync_copy(acc_sh.at[pl.ds(r0, _ACH)],
                            out_hbm.at[cid, pl.ds(r0, _ACH)])
        plsc.subcore_barrier()


def _agg_call(tab1, tab2, send, recv):
    pad = _EPAD - _E
    dummy = _N + (jnp.arange(pad, dtype=jnp.int32) % (_NP - _N))
    send3d = (jnp.arange(_EPAD, dtype=jnp.int32) % _NP).reshape(_NW, _WCH, _ACH)
    recv3d = jnp.concatenate([recv, dummy]).reshape(_NW, _WCH, _ACH)
    f = pl.kernel(
        _agg_body,
        out_type=[jax.ShapeDtypeStruct((_NC, _NP, _D), jnp.float32),
                  jax.ShapeDtypeStruct((_NC, _NP, _D), jnp.float32)],
        mesh=_mesh(),
        scratch_types=[
            pltpu.VMEM((_HNCH, _ACH), jnp.int32),
            pltpu.VMEM((_HNCH, _ACH), jnp.int32),
            pltpu.VMEM((_ACH, _D), jnp.float32),
            pltpu.VMEM((_ACH, _D), jnp.float32),
            pltpu.VMEM_SHARED((_NP, _D), jnp.float32),
            pltpu.SemaphoreType.DMA,
            pltpu.SemaphoreType.DMA,
        ],
    )
    return f(tab1, tab2, send3d, recv3d)


def _feat_body(x_ref, cx_ref, w_ref, b_ref, dsp_ref, h1_ref, h2_ref):
    deg = dsp_ref[0] + dsp_ref[1]
    scale = lax.rsqrt(jnp.maximum(deg, 1.0))[:, None]
    wm = w_ref[...]
    bv = b_ref[...]
    h1_ref[...] = (jnp.dot(x_ref[...], wm,
                           preferred_element_type=jnp.float32) + bv) * scale
    h2_ref[...] = (jnp.dot(cx_ref[...], wm,
                           preferred_element_type=jnp.float32) + bv) * scale


def _feat_call(x_p, cx_p, W, b2, degs_p):
    return pl.pallas_call(
        _feat_body,
        grid=(_GRID,),
        in_specs=[
            pl.BlockSpec((_RB, _D), lambda i: (i, 0)),
            pl.BlockSpec((_RB, _D), lambda i: (i, 0)),
            pl.BlockSpec((_D, _D), lambda i: (0, 0)),
            pl.BlockSpec((1, _D), lambda i: (0, 0)),
            pl.BlockSpec((_NC, _RB), lambda i: (0, i)),
        ],
        out_specs=[
            pl.BlockSpec((_RB, _D), lambda i: (i, 0)),
            pl.BlockSpec((_RB, _D), lambda i: (i, 0)),
        ],
        out_shape=[jax.ShapeDtypeStruct((_NP, _D), jnp.float32),
                   jax.ShapeDtypeStruct((_NP, _D), jnp.float32)],
    )(x_p, cx_p, W, b2, degs_p)


def _selu(v):
    return _SELU_SCALE * jnp.where(v > 0, v, _SELU_ALPHA * (jnp.exp(v) - 1.0))


def _node_body(a1_ref, a2_ref, drp_ref, n1_ref, n2_ref, emb_ref, cs_ref,
               cs_s):
    i = pl.program_id(0)
    agg1 = a1_ref[0] + a1_ref[1]
    agg2 = a2_ref[0] + a2_ref[1]
    deg = drp_ref[0] + drp_ref[1]
    invr = lax.rsqrt(jnp.maximum(deg, 1.0))[:, None]
    n1 = _selu(agg1 * invr)
    n2 = _selu(agg2 * invr)
    n1_ref[...] = n1
    n2_ref[...] = n2
    nrm = jnp.sqrt(jnp.sum(n1 * n1, axis=1, keepdims=True))
    emb_ref[...] = n1 / (nrm + 1e-12)
    bs = jnp.sum(n1, axis=0, keepdims=True)

    @pl.when(i == 0)
    def _():
        cs_s[...] = bs

    @pl.when(i > 0)
    def _():
        cs_s[...] = cs_s[...] + bs

    @pl.when(i == _GRID - 1)
    def _():
        cs_ref[...] = cs_s[...]


def _node_call(agg1_p, agg2_p, degr_p):
    return pl.pallas_call(
        _node_body,
        grid=(_GRID,),
        in_specs=[
            pl.BlockSpec((_NC, _RB, _D), lambda i: (0, i, 0)),
            pl.BlockSpec((_NC, _RB, _D), lambda i: (0, i, 0)),
            pl.BlockSpec((_NC, _RB), lambda i: (0, i)),
        ],
        out_specs=[
            pl.BlockSpec((_RB, _D), lambda i: (i, 0)),
            pl.BlockSpec((_RB, _D), lambda i: (i, 0)),
            pl.BlockSpec((_RB, _D), lambda i: (i, 0)),
            pl.BlockSpec((1, _D), lambda i: (0, 0)),
        ],
        out_shape=[jax.ShapeDtypeStruct((_NP, _D), jnp.float32),
                   jax.ShapeDtypeStruct((_NP, _D), jnp.float32),
                   jax.ShapeDtypeStruct((_NP, _D), jnp.float32),
                   jax.ShapeDtypeStruct((1, _D), jnp.float32)],
        scratch_shapes=[pltpu.VMEM((1, _D), jnp.float32)],
    )(agg1_p, agg2_p, degr_p)


def _head_body(n1_ref, n2_ref, emb_ref, cen_ref, wbl_ref, cs_ref,
               l1_ref, l2_ref, rid_ref, loss_ref,
               rv_s, ri_s, ls_s):
    i = pl.program_id(0)
    cs = cs_ref[0]
    summary = jax.nn.sigmoid(cs * (1.0 / _N))
    v = jnp.sum(wbl_ref[...] * summary[None, :], axis=1)
    n1 = n1_ref[...]
    n2 = n2_ref[...]
    l1_ref[...] = jnp.sum(n1 * v[None, :], axis=1)
    l2_ref[...] = jnp.sum(n2 * v[None, :], axis=1)

    emb = emb_ref[...]
    cen = cen_ref[...]
    e2 = jnp.sum(emb * emb, axis=1)[:, None]
    c2 = jnp.sum(cen * cen, axis=1)[None, :]
    dots = lax.dot_general(emb, cen, (((1,), (1,)), ((), ())),
                           preferred_element_type=jnp.float32)
    d = jnp.sqrt(jnp.maximum(e2 + c2 - 2.0 * dots, 1e-12))
    rowid = lax.broadcasted_iota(jnp.int32, (_RB, _K), 0)
    valid = (rowid + i * _RB) < _N
    dm = jnp.where(valid, d, jnp.inf)
    bmin = jnp.min(dm, axis=0)
    barg = jnp.min(jnp.where(dm == bmin[None, :], rowid, jnp.int32(2 ** 30)),
                   axis=0) + i * _RB
    rmin = jnp.min(dm, axis=1)
    bloss = jnp.sum(jnp.where(valid[:, 0], rmin, 0.0))

    @pl.when(i == 0)
    def _():
        rv_s[...] = bmin
        ri_s[...] = barg
        ls_s[0, 0] = bloss

    @pl.when(i > 0)
    def _():
        better = bmin < rv_s[...]
        rv_s[...] = jnp.where(better, bmin, rv_s[...])
        ri_s[...] = jnp.where(better, barg, ri_s[...])
        ls_s[0, 0] = ls_s[0, 0] + bloss

    @pl.when(i == _GRID - 1)
    def _():
        rid_ref[...] = ri_s[...]
        loss_ref[0, 0] = ls_s[0, 0]


def _head_call(n1, n2, emb_p, centers, w_bl, cs):
    return pl.pallas_call(
        _head_body,
        grid=(_GRID,),
        in_specs=[
            pl.BlockSpec((_RB, _D), lambda i: (i, 0)),
            pl.BlockSpec((_RB, _D), lambda i: (i, 0)),
            pl.BlockSpec((_RB, _D), lambda i: (i, 0)),
            pl.BlockSpec((_K, _D), lambda i: (0, 0)),
            pl.BlockSpec((_D, _D), lambda i: (0, 0)),
            pl.BlockSpec((1, _D), lambda i: (0, 0)),
        ],
        out_specs=[
            pl.BlockSpec((_RB,), lambda i: (i,)),
            pl.BlockSpec((_RB,), lambda i: (i,)),
            pl.BlockSpec((_K,), lambda i: (0,)),
            pl.BlockSpec(memory_space=pltpu.SMEM),
        ],
        out_shape=[jax.ShapeDtypeStruct((_NP,), jnp.float32),
                   jax.ShapeDtypeStruct((_NP,), jnp.float32),
                   jax.ShapeDtypeStruct((_K,), jnp.int32),
                   jax.ShapeDtypeStruct((1, 1), jnp.float32)],
        scratch_shapes=[pltpu.VMEM((_K,), jnp.float32),
                        pltpu.VMEM((_K,), jnp.int32),
                        pltpu.SMEM((1, 1), jnp.float32)],
    )(n1, n2, emb_p, centers, w_bl, cs)


def kernel(x, c_x, edge_index, W, b, w_bl, centers):
    send = edge_index[0].astype(jnp.int32)
    recv = edge_index[1].astype(jnp.int32)
    x_p = jnp.pad(x, ((0, _NP - _N), (0, 0)))
    cx_p = jnp.pad(c_x, ((0, _NP - _N), (0, 0)))

    degs_p, degr_p = _deg_call(send, recv)
    h1, h2 = _feat_call(x_p, cx_p, W, b.reshape(1, _D), degs_p)
    agg1_p, agg2_p = _agg_call(h1, h2, send, recv)
    n1, n2, emb_p, cs = _node_call(agg1_p, agg2_p, degr_p)
    l1, l2, rep_ids, loss = _head_call(n1, n2, emb_p, centers, w_bl, cs)

    logits = jnp.concatenate([l1[:_N], l2[:_N]])
    emb = emb_p[:_N]
    return (emb, centers, rep_ids, loss[0, 0], logits)


# speedup vs baseline: 1.0641x; 1.0641x over previous
import functools

import jax
import jax.numpy as jnp
from jax import lax
from jax.experimental import pallas as pl
from jax.experimental.pallas import tpu as pltpu
from jax.experimental.pallas import tpu_sc as plsc

_NC = 2
_NS = 16
_NW = _NC * _NS

_N = 10000
_NP = 10240
_E = 320000
_D = 128
_K = 512

_RB = 1024
_GRID = _NP // _RB

_EW = _E // _NW
_ECH = 200
_DCH = 2000
_ZROWS = 40

_SELU_ALPHA = 1.6732632423543772
_SELU_SCALE = 1.0507009873554805


def _mesh():
    return plsc.VectorSubcoreMesh(
        core_axis_name="c", subcore_axis_name="s",
        num_cores=_NC, num_subcores=_NS)


def _deg_body(send_hbm, recv_hbm, outs_hbm, outr_hbm,
              idx_v, ones_v, zb_v, hs_sh, hr_sh):
    cid = lax.axis_index("c")
    sid = lax.axis_index("s")
    w = cid * _NS + sid

    def fill_ones(i, carry):
        ones_v[pl.ds(i * 16, 16)] = jnp.full((16,), 1.0, jnp.float32)
        return carry
    lax.fori_loop(0, _DCH // 16, fill_ones, 0)

    def fill_zero(i, carry):
        zb_v[pl.ds(i * 16, 16)] = jnp.zeros((16,), jnp.float32)
        return carry
    lax.fori_loop(0, 640 // 16, fill_zero, 0)

    pltpu.sync_copy(zb_v, hs_sh.at[pl.ds(sid * 640, 640)])
    pltpu.sync_copy(zb_v, hr_sh.at[pl.ds(sid * 640, 640)])
    plsc.subcore_barrier()

    def step(i, carry):
        base = w * _EW + i * _DCH
        pltpu.sync_copy(send_hbm.at[pl.ds(base, _DCH)], idx_v)
        pltpu.sync_copy(ones_v, hs_sh.at[idx_v], add=True)
        pltpu.sync_copy(recv_hbm.at[pl.ds(base, _DCH)], idx_v)
        pltpu.sync_copy(ones_v, hr_sh.at[idx_v], add=True)
        return carry
    lax.fori_loop(0, _EW // _DCH, step, 0)
    plsc.subcore_barrier()

    pltpu.sync_copy(hs_sh.at[pl.ds(sid * 640, 640)],
                    outs_hbm.at[cid, pl.ds(sid * 640, 640)])
    pltpu.sync_copy(hr_sh.at[pl.ds(sid * 640, 640)],
                    outr_hbm.at[cid, pl.ds(sid * 640, 640)])


def _deg_call(send, recv):
    f = pl.kernel(
        _deg_body,
        out_type=[jax.ShapeDtypeStruct((_NC, _NP), jnp.float32),
                  jax.ShapeDtypeStruct((_NC, _NP), jnp.float32)],
        mesh=_mesh(),
        scratch_types=[
            pltpu.VMEM((_DCH,), jnp.int32),
            pltpu.VMEM((_DCH,), jnp.float32),
            pltpu.VMEM((640,), jnp.float32),
            pltpu.VMEM_SHARED((_NP,), jnp.float32),
            pltpu.VMEM_SHARED((_NP,), jnp.float32),
        ],
    )
    return f(send, recv)


_ACH = 128
_WCH = 80
_HNCH = 40
_EPAD = _NW * _WCH * _ACH


def _agg_body(tab1_hbm, tab2_hbm, send3d_hbm, recv3d_hbm, out1_hbm, out2_hbm,
              sidx_v, ridx_v, buf0, buf1, acc_sh, sem0, sem1):
    cid = lax.axis_index("c")
    sid = lax.axis_index("s")
    w = cid * _NS + sid

    for tab_hbm, out_hbm in ((tab1_hbm, out1_hbm), (tab2_hbm, out2_hbm)):
        def zrow(i, carry):
            for cix in range(_D // 16):
                buf0[i, pl.ds(cix * 16, 16)] = jnp.zeros((16,), jnp.float32)
            return carry
        lax.fori_loop(0, _ACH, zrow, 0)
        for j in range(640 // _ACH):
            pltpu.sync_copy(buf0,
                            acc_sh.at[pl.ds(sid * 640 + j * _ACH, _ACH)])
        plsc.subcore_barrier()

        def gstart(i, buf, sem):
            pltpu.make_async_copy(tab_hbm.at[sidx_v.at[i]], buf, sem).start()

        def gwait(i, buf, sem):
            pltpu.make_async_copy(tab_hbm.at[sidx_v.at[i]], buf, sem).wait()

        for h in range(2):
            pltpu.sync_copy(send3d_hbm.at[w, pl.ds(h * _HNCH, _HNCH)], sidx_v)
            pltpu.sync_copy(recv3d_hbm.at[w, pl.ds(h * _HNCH, _HNCH)], ridx_v)

            gstart(0, buf0, sem0)

            def step(i, carry):
                @pl.when(i % 2 == 0)
                def _():
                    gwait(i, buf0, sem0)

                    @pl.when(i + 1 < _HNCH)
                    def _():
                        gstart(i + 1, buf1, sem1)
                    pltpu.sync_copy(buf0, acc_sh.at[ridx_v.at[i]], add=True)

                @pl.when(i % 2 == 1)
                def _():
                    gwait(i, buf1, sem1)

                    @pl.when(i + 1 < _HNCH)
                    def _():
                        gstart(i + 1, buf0, sem0)
                    pltpu.sync_copy(buf1, acc_sh.at[ridx_v.at[i]], add=True)
                return carry
            lax.fori_loop(0, _HNCH, step, 0)
        plsc.subcore_barrier()

        for j in range(640 // _ACH):
            r0 = sid * 640 + j * _ACH
            pltpu.sync_copy(acc_sh.at[pl.ds(r0, _ACH)],
                            out_hbm.at[cid, pl.ds(r0, _ACH)])
        plsc.subcore_barrier()


def _agg_call(tab1, tab2, send, recv):
    pad = _EPAD - _E
    spread = jnp.arange(pad, dtype=jnp.int32) % (_NP - _N)
    send3d = jnp.concatenate([send, spread]).reshape(_NW, _WCH, _ACH)
    recv3d = jnp.concatenate([recv, _N + spread]).reshape(_NW, _WCH, _ACH)
    f = pl.kernel(
        _agg_body,
        out_type=[jax.ShapeDtypeStruct((_NC, _NP, _D), jnp.float32),
                  jax.ShapeDtypeStruct((_NC, _NP, _D), jnp.float32)],
        mesh=_mesh(),
        scratch_types=[
            pltpu.VMEM((_HNCH, _ACH), jnp.int32),
            pltpu.VMEM((_HNCH, _ACH), jnp.int32),
            pltpu.VMEM((_ACH, _D), jnp.float32),
            pltpu.VMEM((_ACH, _D), jnp.float32),
            pltpu.VMEM_SHARED((_NP, _D), jnp.float32),
            pltpu.SemaphoreType.DMA,
            pltpu.SemaphoreType.DMA,
        ],
    )
    return f(tab1, tab2, send3d, recv3d)


def _feat_body(x_ref, cx_ref, w_ref, b_ref, dsp_ref, h1_ref, h2_ref):
    deg = dsp_ref[0] + dsp_ref[1]
    scale = lax.rsqrt(jnp.maximum(deg, 1.0))[:, None]
    wm = w_ref[...]
    bv = b_ref[...]
    h1_ref[...] = (jnp.dot(x_ref[...], wm,
                           preferred_element_type=jnp.float32) + bv) * scale
    h2_ref[...] = (jnp.dot(cx_ref[...], wm,
                           preferred_element_type=jnp.float32) + bv) * scale


def _feat_call(x, c_x, W, b2, degs_p):
    return pl.pallas_call(
        _feat_body,
        grid=(_GRID,),
        in_specs=[
            pl.BlockSpec((_RB, _D), lambda i: (i, 0)),
            pl.BlockSpec((_RB, _D), lambda i: (i, 0)),
            pl.BlockSpec((_D, _D), lambda i: (0, 0)),
            pl.BlockSpec((1, _D), lambda i: (0, 0)),
            pl.BlockSpec((_NC, _RB), lambda i: (0, i)),
        ],
        out_specs=[
            pl.BlockSpec((_RB, _D), lambda i: (i, 0)),
            pl.BlockSpec((_RB, _D), lambda i: (i, 0)),
        ],
        out_shape=[jax.ShapeDtypeStruct((_N, _D), jnp.float32),
                   jax.ShapeDtypeStruct((_N, _D), jnp.float32)],
    )(x, c_x, W, b2, degs_p)


def _selu(v):
    return _SELU_SCALE * jnp.where(v > 0, v, _SELU_ALPHA * (jnp.exp(v) - 1.0))


def _node_body(a1_ref, a2_ref, drp_ref, n1_ref, n2_ref, emb_ref, cs_ref,
               cs_s):
    i = pl.program_id(0)
    agg1 = a1_ref[0] + a1_ref[1]
    agg2 = a2_ref[0] + a2_ref[1]
    deg = drp_ref[0] + drp_ref[1]
    invr = lax.rsqrt(jnp.maximum(deg, 1.0))[:, None]
    n1 = _selu(agg1 * invr)
    n2 = _selu(agg2 * invr)
    n1_ref[...] = n1.astype(jnp.bfloat16)
    n2_ref[...] = n2.astype(jnp.bfloat16)
    nrm = jnp.sqrt(jnp.sum(n1 * n1, axis=1, keepdims=True))
    emb_ref[...] = n1 / (nrm + 1e-12)
    vrow = (lax.broadcasted_iota(jnp.int32, (_RB, 1), 0) + i * _RB) < _N
    bs = jnp.sum(jnp.where(vrow, n1, 0.0), axis=0, keepdims=True)

    @pl.when(i == 0)
    def _():
        cs_s[...] = bs

    @pl.when(i > 0)
    def _():
        cs_s[...] = cs_s[...] + bs

    @pl.when(i == _GRID - 1)
    def _():
        cs_ref[...] = cs_s[...]


def _node_call(agg1_p, agg2_p, degr_p):
    return pl.pallas_call(
        _node_body,
        grid=(_GRID,),
        in_specs=[
            pl.BlockSpec((_NC, _RB, _D), lambda i: (0, i, 0)),
            pl.BlockSpec((_NC, _RB, _D), lambda i: (0, i, 0)),
            pl.BlockSpec((_NC, _RB), lambda i: (0, i)),
        ],
        out_specs=[
            pl.BlockSpec((_RB, _D), lambda i: (i, 0)),
            pl.BlockSpec((_RB, _D), lambda i: (i, 0)),
            pl.BlockSpec((_RB, _D), lambda i: (i, 0)),
            pl.BlockSpec((1, _D), lambda i: (0, 0)),
        ],
        out_shape=[jax.ShapeDtypeStruct((_N, _D), jnp.bfloat16),
                   jax.ShapeDtypeStruct((_N, _D), jnp.bfloat16),
                   jax.ShapeDtypeStruct((_N, _D), jnp.float32),
                   jax.ShapeDtypeStruct((1, _D), jnp.float32)],
        scratch_shapes=[pltpu.VMEM((1, _D), jnp.float32)],
    )(agg1_p, agg2_p, degr_p)


def _head_body(n1_ref, n2_ref, emb_ref, cen_ref, wbl_ref, cs_ref,
               l1_ref, l2_ref, rid_ref, loss_ref,
               rv_s, ri_s, ls_s):
    i = pl.program_id(0)
    cs = cs_ref[0]
    summary = jax.nn.sigmoid(cs * (1.0 / _N))
    v = jnp.sum(wbl_ref[...] * summary[None, :], axis=1)
    n1 = n1_ref[...].astype(jnp.float32)
    n2 = n2_ref[...].astype(jnp.float32)
    l1_ref[...] = jnp.sum(n1 * v[None, :], axis=1)
    l2_ref[...] = jnp.sum(n2 * v[None, :], axis=1)

    emb = emb_ref[...]
    cen = cen_ref[...]
    e2 = jnp.sum(emb * emb, axis=1)[:, None]
    c2 = jnp.sum(cen * cen, axis=1)[None, :]
    dots = lax.dot_general(emb, cen, (((1,), (1,)), ((), ())),
                           preferred_element_type=jnp.float32)
    d = jnp.sqrt(jnp.maximum(e2 + c2 - 2.0 * dots, 1e-12))
    rowid = lax.broadcasted_iota(jnp.int32, (_RB, _K), 0)
    valid = (rowid + i * _RB) < _N
    dm = jnp.where(valid, d, jnp.inf)
    bmin = jnp.min(dm, axis=0)
    barg = jnp.min(jnp.where(dm == bmin[None, :], rowid, jnp.int32(2 ** 30)),
                   axis=0) + i * _RB
    rmin = jnp.min(dm, axis=1)
    bloss = jnp.sum(jnp.where(valid[:, 0], rmin, 0.0))

    @pl.when(i == 0)
    def _():
        rv_s[...] = bmin
        ri_s[...] = barg
        ls_s[0, 0] = bloss

    @pl.when(i > 0)
    def _():
        better = bmin < rv_s[...]
        rv_s[...] = jnp.where(better, bmin, rv_s[...])
        ri_s[...] = jnp.where(better, barg, ri_s[...])
        ls_s[0, 0] = ls_s[0, 0] + bloss

    @pl.when(i == _GRID - 1)
    def _():
        rid_ref[...] = ri_s[...]
        loss_ref[0, 0] = ls_s[0, 0]


def _head_call(n1, n2, emb_p, centers, w_bl, cs):
    return pl.pallas_call(
        _head_body,
        grid=(_GRID,),
        in_specs=[
            pl.BlockSpec((_RB, _D), lambda i: (i, 0)),
            pl.BlockSpec((_RB, _D), lambda i: (i, 0)),
            pl.BlockSpec((_RB, _D), lambda i: (i, 0)),
            pl.BlockSpec((_K, _D), lambda i: (0, 0)),
            pl.BlockSpec((_D, _D), lambda i: (0, 0)),
            pl.BlockSpec((1, _D), lambda i: (0, 0)),
        ],
        out_specs=[
            pl.BlockSpec((_RB,), lambda i: (i,)),
            pl.BlockSpec((_RB,), lambda i: (i,)),
            pl.BlockSpec((_K,), lambda i: (0,)),
            pl.BlockSpec(memory_space=pltpu.SMEM),
        ],
        out_shape=[jax.ShapeDtypeStruct((_N,), jnp.float32),
                   jax.ShapeDtypeStruct((_N,), jnp.float32),
                   jax.ShapeDtypeStruct((_K,), jnp.int32),
                   jax.ShapeDtypeStruct((1, 1), jnp.float32)],
        scratch_shapes=[pltpu.VMEM((_K,), jnp.float32),
                        pltpu.VMEM((_K,), jnp.int32),
                        pltpu.SMEM((1, 1), jnp.float32)],
    )(n1, n2, emb_p, centers, w_bl, cs)


def kernel(x, c_x, edge_index, W, b, w_bl, centers):
    send = edge_index[0].astype(jnp.int32)
    recv = edge_index[1].astype(jnp.int32)

    degs_p, degr_p = _deg_call(send, recv)
    h1, h2 = _feat_call(x, c_x, W, b.reshape(1, _D), degs_p)
    agg1_p, agg2_p = _agg_call(h1, h2, send, recv)
    n1, n2, emb, cs = _node_call(agg1_p, agg2_p, degr_p)
    l1, l2, rep_ids, loss = _head_call(n1, n2, emb, centers, w_bl, cs)

    logits = jnp.concatenate([l1, l2])
    return (emb, centers, rep_ids, loss[0, 0], logits)


# speedup vs baseline: 1.1031x; 1.0367x over previous
import functools

import jax
import jax.numpy as jnp
from jax import lax
from jax.experimental import pallas as pl
from jax.experimental.pallas import tpu as pltpu
from jax.experimental.pallas import tpu_sc as plsc

_NC = 2
_NS = 16
_NW = _NC * _NS

_N = 10000
_NP = 10240
_E = 320000
_D = 128
_K = 512

_RB = 1024
_GRID = _NP // _RB

_EW = _E // _NW
_ECH = 200
_DCH = 2000
_ZROWS = 40

_SELU_ALPHA = 1.6732632423543772
_SELU_SCALE = 1.0507009873554805


def _mesh():
    return plsc.VectorSubcoreMesh(
        core_axis_name="c", subcore_axis_name="s",
        num_cores=_NC, num_subcores=_NS)


def _deg_body(edges_hbm, outs_hbm, outr_hbm,
              idx_v, ones_v, zb_v, hs_sh, hr_sh):
    cid = lax.axis_index("c")
    sid = lax.axis_index("s")
    w = cid * _NS + sid

    def fill_ones(i, carry):
        ones_v[pl.ds(i * 16, 16)] = jnp.full((16,), 1.0, jnp.float32)
        return carry
    lax.fori_loop(0, _DCH // 16, fill_ones, 0)

    def fill_zero(i, carry):
        zb_v[pl.ds(i * 16, 16)] = jnp.zeros((16,), jnp.float32)
        return carry
    lax.fori_loop(0, 640 // 16, fill_zero, 0)

    pltpu.sync_copy(zb_v, hs_sh.at[pl.ds(sid * 640, 640)])
    pltpu.sync_copy(zb_v, hr_sh.at[pl.ds(sid * 640, 640)])
    plsc.subcore_barrier()

    def step(i, carry):
        base = w * _EW + i * _DCH
        pltpu.sync_copy(edges_hbm.at[pl.ds(base, _DCH)], idx_v)
        pltpu.sync_copy(ones_v, hs_sh.at[idx_v], add=True)
        pltpu.sync_copy(edges_hbm.at[pl.ds(_E + base, _DCH)], idx_v)
        pltpu.sync_copy(ones_v, hr_sh.at[idx_v], add=True)
        return carry
    lax.fori_loop(0, _EW // _DCH, step, 0)
    plsc.subcore_barrier()

    pltpu.sync_copy(hs_sh.at[pl.ds(sid * 640, 640)],
                    outs_hbm.at[cid, pl.ds(sid * 640, 640)])
    pltpu.sync_copy(hr_sh.at[pl.ds(sid * 640, 640)],
                    outr_hbm.at[cid, pl.ds(sid * 640, 640)])


def _deg_call(edges_flat):
    f = pl.kernel(
        _deg_body,
        out_type=[jax.ShapeDtypeStruct((_NC, _NP), jnp.float32),
                  jax.ShapeDtypeStruct((_NC, _NP), jnp.float32)],
        mesh=_mesh(),
        scratch_types=[
            pltpu.VMEM((_DCH,), jnp.int32),
            pltpu.VMEM((_DCH,), jnp.float32),
            pltpu.VMEM((640,), jnp.float32),
            pltpu.VMEM_SHARED((_NP,), jnp.float32),
            pltpu.VMEM_SHARED((_NP,), jnp.float32),
        ],
    )
    return f(edges_flat)


_ACH = 128
_WCH = 80
_HNCH = 40
_EPAD = _NW * _WCH * _ACH


def _agg_body(tab1_hbm, tab2_hbm, send3d_hbm, recv3d_hbm, out1_hbm, out2_hbm,
              sidx_v, ridx_v, buf0, buf1, acc_sh, sem0, sem1):
    cid = lax.axis_index("c")
    sid = lax.axis_index("s")
    w = cid * _NS + sid

    for tab_hbm, out_hbm in ((tab1_hbm, out1_hbm), (tab2_hbm, out2_hbm)):
        def zrow(i, carry):
            for cix in range(_D // 16):
                buf0[i, pl.ds(cix * 16, 16)] = jnp.zeros((16,), jnp.float32)
            return carry
        lax.fori_loop(0, _ACH, zrow, 0)
        for j in range(640 // _ACH):
            pltpu.sync_copy(buf0,
                            acc_sh.at[pl.ds(sid * 640 + j * _ACH, _ACH)])
        plsc.subcore_barrier()

        def gstart(i, buf, sem):
            pltpu.make_async_copy(tab_hbm.at[sidx_v.at[i]], buf, sem).start()

        def gwait(i, buf, sem):
            pltpu.make_async_copy(tab_hbm.at[sidx_v.at[i]], buf, sem).wait()

        for h in range(2):
            pltpu.sync_copy(send3d_hbm.at[w, pl.ds(h * _HNCH, _HNCH)], sidx_v)
            pltpu.sync_copy(recv3d_hbm.at[w, pl.ds(h * _HNCH, _HNCH)], ridx_v)

            gstart(0, buf0, sem0)

            def step(i, carry):
                @pl.when(i % 2 == 0)
                def _():
                    gwait(i, buf0, sem0)

                    @pl.when(i + 1 < _HNCH)
                    def _():
                        gstart(i + 1, buf1, sem1)
                    pltpu.sync_copy(buf0, acc_sh.at[ridx_v.at[i]], add=True)

                @pl.when(i % 2 == 1)
                def _():
                    gwait(i, buf1, sem1)

                    @pl.when(i + 1 < _HNCH)
                    def _():
                        gstart(i + 1, buf0, sem0)
                    pltpu.sync_copy(buf1, acc_sh.at[ridx_v.at[i]], add=True)
                return carry
            lax.fori_loop(0, _HNCH, step, 0)
        plsc.subcore_barrier()

        for j in range(640 // _ACH):
            r0 = sid * 640 + j * _ACH
            pltpu.sync_copy(acc_sh.at[pl.ds(r0, _ACH)],
                            out_hbm.at[cid, pl.ds(r0, _ACH)])
        plsc.subcore_barrier()


def _agg_call(tab1, tab2, send, recv):
    pad = _EPAD - _E
    spread = jnp.arange(pad, dtype=jnp.int32) % (_NP - _N)
    send3d = jnp.concatenate([send, spread]).reshape(_NW, _WCH, _ACH)
    recv3d = jnp.concatenate([recv, _N + spread]).reshape(_NW, _WCH, _ACH)
    f = pl.kernel(
        _agg_body,
        out_type=[jax.ShapeDtypeStruct((_NC, _NP, _D), jnp.float32),
                  jax.ShapeDtypeStruct((_NC, _NP, _D), jnp.float32)],
        mesh=_mesh(),
        scratch_types=[
            pltpu.VMEM((_HNCH, _ACH), jnp.int32),
            pltpu.VMEM((_HNCH, _ACH), jnp.int32),
            pltpu.VMEM((_ACH, _D), jnp.float32),
            pltpu.VMEM((_ACH, _D), jnp.float32),
            pltpu.VMEM_SHARED((_NP, _D), jnp.float32),
            pltpu.SemaphoreType.DMA,
            pltpu.SemaphoreType.DMA,
        ],
    )
    return f(tab1, tab2, send3d, recv3d)


def _feat_body(x_ref, cx_ref, w_ref, b_ref, dsp_ref, h1_ref, h2_ref):
    deg = dsp_ref[0] + dsp_ref[1]
    scale = lax.rsqrt(jnp.maximum(deg, 1.0))[:, None]
    wm = w_ref[...]
    bv = b_ref[...]
    h1_ref[...] = (jnp.dot(x_ref[...], wm,
                           preferred_element_type=jnp.float32) + bv) * scale
    h2_ref[...] = (jnp.dot(cx_ref[...], wm,
                           preferred_element_type=jnp.float32) + bv) * scale


def _feat_call(x, c_x, W, b2, degs_p):
    return pl.pallas_call(
        _feat_body,
        grid=(_GRID,),
        in_specs=[
            pl.BlockSpec((_RB, _D), lambda i: (i, 0)),
            pl.BlockSpec((_RB, _D), lambda i: (i, 0)),
            pl.BlockSpec((_D, _D), lambda i: (0, 0)),
            pl.BlockSpec((1, _D), lambda i: (0, 0)),
            pl.BlockSpec((_NC, _RB), lambda i: (0, i)),
        ],
        out_specs=[
            pl.BlockSpec((_RB, _D), lambda i: (i, 0)),
            pl.BlockSpec((_RB, _D), lambda i: (i, 0)),
        ],
        out_shape=[jax.ShapeDtypeStruct((_N, _D), jnp.float32),
                   jax.ShapeDtypeStruct((_N, _D), jnp.float32)],
    )(x, c_x, W, b2, degs_p)


def _selu(v):
    return _SELU_SCALE * jnp.where(v > 0, v, _SELU_ALPHA * (jnp.exp(v) - 1.0))


def _node_body(a1_ref, a2_ref, drp_ref, n1_ref, n2_ref, emb_ref, cs_ref,
               cs_s):
    i = pl.program_id(0)
    agg1 = a1_ref[0] + a1_ref[1]
    agg2 = a2_ref[0] + a2_ref[1]
    deg = drp_ref[0] + drp_ref[1]
    invr = lax.rsqrt(jnp.maximum(deg, 1.0))[:, None]
    n1 = _selu(agg1 * invr)
    n2 = _selu(agg2 * invr)
    n1_ref[...] = n1.astype(jnp.bfloat16)
    n2_ref[...] = n2.astype(jnp.bfloat16)
    nrm = jnp.sqrt(jnp.sum(n1 * n1, axis=1, keepdims=True))
    emb_ref[...] = n1 / (nrm + 1e-12)
    vrow = (lax.broadcasted_iota(jnp.int32, (_RB, 1), 0) + i * _RB) < _N
    bs = jnp.sum(jnp.where(vrow, n1, 0.0), axis=0, keepdims=True)

    @pl.when(i == 0)
    def _():
        cs_s[...] = bs

    @pl.when(i > 0)
    def _():
        cs_s[...] = cs_s[...] + bs

    @pl.when(i == _GRID - 1)
    def _():
        cs_ref[...] = cs_s[...]


def _node_call(agg1_p, agg2_p, degr_p):
    return pl.pallas_call(
        _node_body,
        grid=(_GRID,),
        in_specs=[
            pl.BlockSpec((_NC, _RB, _D), lambda i: (0, i, 0)),
            pl.BlockSpec((_NC, _RB, _D), lambda i: (0, i, 0)),
            pl.BlockSpec((_NC, _RB), lambda i: (0, i)),
        ],
        out_specs=[
            pl.BlockSpec((_RB, _D), lambda i: (i, 0)),
            pl.BlockSpec((_RB, _D), lambda i: (i, 0)),
            pl.BlockSpec((_RB, _D), lambda i: (i, 0)),
            pl.BlockSpec((1, _D), lambda i: (0, 0)),
        ],
        out_shape=[jax.ShapeDtypeStruct((_N, _D), jnp.bfloat16),
                   jax.ShapeDtypeStruct((_N, _D), jnp.bfloat16),
                   jax.ShapeDtypeStruct((_N, _D), jnp.float32),
                   jax.ShapeDtypeStruct((1, _D), jnp.float32)],
        scratch_shapes=[pltpu.VMEM((1, _D), jnp.float32)],
    )(agg1_p, agg2_p, degr_p)


def _head_body(n1_ref, n2_ref, emb_ref, cen_ref, wbl_ref, cs_ref,
               l1_ref, l2_ref, rid_ref, loss_ref,
               rv_s, ri_s, ls_s):
    i = pl.program_id(0)
    cs = cs_ref[0]
    summary = jax.nn.sigmoid(cs * (1.0 / _N))
    v = jnp.sum(wbl_ref[...] * summary[None, :], axis=1)
    n1 = n1_ref[...].astype(jnp.float32)
    n2 = n2_ref[...].astype(jnp.float32)
    l1_ref[...] = jnp.dot(n1, v[:, None],
                          preferred_element_type=jnp.float32)[:, 0]
    l2_ref[...] = jnp.dot(n2, v[:, None],
                          preferred_element_type=jnp.float32)[:, 0]

    emb = emb_ref[...]
    cen = cen_ref[...]
    e2 = jnp.sum(emb * emb, axis=1)[:, None]
    c2 = jnp.sum(cen * cen, axis=1)[None, :]
    dots = lax.dot_general(emb, cen, (((1,), (1,)), ((), ())),
                           preferred_element_type=jnp.float32)
    sq = e2 + c2 - 2.0 * dots
    rowid = lax.broadcasted_iota(jnp.int32, (_RB, _K), 0)
    valid = (rowid + i * _RB) < _N
    dm = jnp.where(valid, sq, jnp.inf)
    bmin = jnp.min(dm, axis=0)
    barg = jnp.min(jnp.where(dm == bmin[None, :], rowid, jnp.int32(2 ** 30)),
                   axis=0) + i * _RB
    rmin = jnp.sqrt(jnp.maximum(jnp.min(dm, axis=1), 1e-12))
    bloss = jnp.sum(jnp.where(valid[:, 0], rmin, 0.0))

    @pl.when(i == 0)
    def _():
        rv_s[...] = bmin
        ri_s[...] = barg
        ls_s[0, 0] = bloss

    @pl.when(i > 0)
    def _():
        better = bmin < rv_s[...]
        rv_s[...] = jnp.where(better, bmin, rv_s[...])
        ri_s[...] = jnp.where(better, barg, ri_s[...])
        ls_s[0, 0] = ls_s[0, 0] + bloss

    @pl.when(i == _GRID - 1)
    def _():
        rid_ref[...] = ri_s[...]
        loss_ref[0, 0] = ls_s[0, 0]


def _head_call(n1, n2, emb_p, centers, w_bl, cs):
    return pl.pallas_call(
        _head_body,
        grid=(_GRID,),
        in_specs=[
            pl.BlockSpec((_RB, _D), lambda i: (i, 0)),
            pl.BlockSpec((_RB, _D), lambda i: (i, 0)),
            pl.BlockSpec((_RB, _D), lambda i: (i, 0)),
            pl.BlockSpec((_K, _D), lambda i: (0, 0)),
            pl.BlockSpec((_D, _D), lambda i: (0, 0)),
            pl.BlockSpec((1, _D), lambda i: (0, 0)),
        ],
        out_specs=[
            pl.BlockSpec((_RB,), lambda i: (i,)),
            pl.BlockSpec((_RB,), lambda i: (i,)),
            pl.BlockSpec((_K,), lambda i: (0,)),
            pl.BlockSpec(memory_space=pltpu.SMEM),
        ],
        out_shape=[jax.ShapeDtypeStruct((_N,), jnp.float32),
                   jax.ShapeDtypeStruct((_N,), jnp.float32),
                   jax.ShapeDtypeStruct((_K,), jnp.int32),
                   jax.ShapeDtypeStruct((1, 1), jnp.float32)],
        scratch_shapes=[pltpu.VMEM((_K,), jnp.float32),
                        pltpu.VMEM((_K,), jnp.int32),
                        pltpu.SMEM((1, 1), jnp.float32)],
    )(n1, n2, emb_p, centers, w_bl, cs)


def kernel(x, c_x, edge_index, W, b, w_bl, centers):
    edges_flat = edge_index.astype(jnp.int32).reshape(-1)
    send = edges_flat[:_E]
    recv = edges_flat[_E:]

    degs_p, degr_p = _deg_call(edges_flat)
    h1, h2 = _feat_call(x, c_x, W, b.reshape(1, _D), degs_p)
    agg1_p, agg2_p = _agg_call(h1, h2, send, recv)
    n1, n2, emb, cs = _node_call(agg1_p, agg2_p, degr_p)
    l1, l2, rep_ids, loss = _head_call(n1, n2, emb, centers, w_bl, cs)

    logits = jnp.concatenate([l1, l2])
    return (emb, centers, rep_ids, loss[0, 0], logits)


# speedup vs baseline: 1.2599x; 1.1421x over previous
import functools

import jax
import jax.numpy as jnp
from jax import lax
from jax.experimental import pallas as pl
from jax.experimental.pallas import tpu as pltpu
from jax.experimental.pallas import tpu_sc as plsc

_NC = 2
_NS = 16
_NW = _NC * _NS

_N = 10000
_NP = 10240
_E = 320000
_D = 128
_K = 512

_RB = 1024
_GRID = _NP // _RB

_EW = _E // _NW
_ECH = 200
_DCH = 2000
_ZROWS = 40

_SELU_ALPHA = 1.6732632423543772
_SELU_SCALE = 1.0507009873554805


def _mesh():
    return plsc.VectorSubcoreMesh(
        core_axis_name="c", subcore_axis_name="s",
        num_cores=_NC, num_subcores=_NS)


def _deg_body(edges_hbm, outs_hbm, outr_hbm,
              idx_v, ones_v, zb_v, hs_sh, hr_sh):
    cid = lax.axis_index("c")
    sid = lax.axis_index("s")
    w = cid * _NS + sid

    def fill_ones(i, carry):
        ones_v[pl.ds(i * 16, 16)] = jnp.full((16,), 1.0, jnp.float32)
        return carry
    lax.fori_loop(0, _DCH // 16, fill_ones, 0)

    def fill_zero(i, carry):
        zb_v[pl.ds(i * 16, 16)] = jnp.zeros((16,), jnp.float32)
        return carry
    lax.fori_loop(0, 640 // 16, fill_zero, 0)

    pltpu.sync_copy(zb_v, hs_sh.at[pl.ds(sid * 640, 640)])
    pltpu.sync_copy(zb_v, hr_sh.at[pl.ds(sid * 640, 640)])
    plsc.subcore_barrier()

    def step(i, carry):
        base = w * _EW + i * _DCH
        pltpu.sync_copy(edges_hbm.at[pl.ds(base, _DCH)], idx_v)
        pltpu.sync_copy(ones_v, hs_sh.at[idx_v], add=True)
        pltpu.sync_copy(edges_hbm.at[pl.ds(_E + base, _DCH)], idx_v)
        pltpu.sync_copy(ones_v, hr_sh.at[idx_v], add=True)
        return carry
    lax.fori_loop(0, _EW // _DCH, step, 0)
    plsc.subcore_barrier()

    pltpu.sync_copy(hs_sh.at[pl.ds(sid * 640, 640)],
                    outs_hbm.at[cid, pl.ds(sid * 640, 640)])
    pltpu.sync_copy(hr_sh.at[pl.ds(sid * 640, 640)],
                    outr_hbm.at[cid, pl.ds(sid * 640, 640)])


def _deg_call(edges_flat):
    f = pl.kernel(
        _deg_body,
        out_type=[jax.ShapeDtypeStruct((_NC, _NP), jnp.float32),
                  jax.ShapeDtypeStruct((_NC, _NP), jnp.float32)],
        mesh=_mesh(),
        scratch_types=[
            pltpu.VMEM((_DCH,), jnp.int32),
            pltpu.VMEM((_DCH,), jnp.float32),
            pltpu.VMEM((640,), jnp.float32),
            pltpu.VMEM_SHARED((_NP,), jnp.float32),
            pltpu.VMEM_SHARED((_NP,), jnp.float32),
        ],
    )
    return f(edges_flat)


_ACH = 88
_WCH = 120
_TNCH = 40
_EPAD = _NW * _WCH * _ACH


def _agg_body(tab1_hbm, tab2_hbm, send3d_hbm, recv3d_hbm, out1_hbm, out2_hbm,
              sidx_v, ridx_v, buf0, buf1, buf2,
              acc_sh, gs0, gs1, gs2, ss0, ss1):
    cid = lax.axis_index("c")
    sid = lax.axis_index("s")
    w = cid * _NS + sid
    bufs = (buf0, buf1, buf2)
    gsems = (gs0, gs1, gs2)
    ssems = (ss0, ss1)

    for tab_hbm, out_hbm in ((tab1_hbm, out1_hbm), (tab2_hbm, out2_hbm)):
        def zrow(i, carry):
            for cix in range(_D // 16):
                buf0[i, pl.ds(cix * 16, 16)] = jnp.zeros((16,), jnp.float32)
            return carry
        lax.fori_loop(0, _ACH, zrow, 0)
        nz = (640 + _ACH - 1) // _ACH
        for j in range(nz):
            r0 = sid * 640 + j * _ACH
            rows = min(_ACH, 640 - j * _ACH)
            pltpu.sync_copy(buf0.at[pl.ds(0, rows)],
                            acc_sh.at[pl.ds(r0, rows)])
        plsc.subcore_barrier()

        def gstart(i, buf, sem):
            pltpu.make_async_copy(tab_hbm.at[sidx_v.at[i]], buf, sem).start()

        def gwait(i, buf, sem):
            pltpu.make_async_copy(tab_hbm.at[sidx_v.at[i]], buf, sem).wait()

        def sstart(i, buf, sem):
            pltpu.make_async_copy(buf, acc_sh.at[ridx_v.at[i]],
                                  sem).start(add=True)

        def swait(i, buf, sem):
            pltpu.make_async_copy(buf, acc_sh.at[ridx_v.at[i]], sem).wait()

        for t in range(_WCH // _TNCH):
            pltpu.sync_copy(send3d_hbm.at[w, pl.ds(t * _TNCH, _TNCH)], sidx_v)
            pltpu.sync_copy(recv3d_hbm.at[w, pl.ds(t * _TNCH, _TNCH)], ridx_v)

            gstart(0, buf0, gs0)
            gstart(1, buf1, gs1)

            def step(i, carry):
                for k in range(6):
                    @pl.when(i % 6 == k)
                    def _(k=k):
                        b = k % 3
                        s = k % 2
                        gwait(i, bufs[b], gsems[b])
                        sstart(i, bufs[b], ssems[s])

                        @pl.when(i == 0)
                        def _():
                            gstart(i + 2, bufs[(b + 2) % 3],
                                   gsems[(b + 2) % 3])

                        @pl.when((i >= 1) & (i + 2 < _TNCH))
                        def _():
                            swait(i - 1, bufs[(b + 2) % 3], ssems[1 - s])
                            gstart(i + 2, bufs[(b + 2) % 3],
                                   gsems[(b + 2) % 3])
                return carry
            lax.fori_loop(0, _TNCH, step, 0)
            swait(_TNCH - 3, bufs[(_TNCH - 3) % 3], ssems[(_TNCH - 3) % 2])
            swait(_TNCH - 2, bufs[(_TNCH - 2) % 3], ssems[(_TNCH - 2) % 2])
            swait(_TNCH - 1, bufs[(_TNCH - 1) % 3], ssems[(_TNCH - 1) % 2])
        plsc.subcore_barrier()

        nz = (640 + _ACH - 1) // _ACH
        for j in range(nz):
            r0 = sid * 640 + j * _ACH
            rows = min(_ACH, 640 - j * _ACH)
            pltpu.sync_copy(acc_sh.at[pl.ds(r0, rows)],
                            out_hbm.at[cid, pl.ds(r0, rows)])
        plsc.subcore_barrier()


def _agg_call(tab1, tab2, send, recv):
    pad = _EPAD - _E
    spread = jnp.arange(pad, dtype=jnp.int32) % (_NP - _N)
    send3d = jnp.concatenate([send, spread]).reshape(_NW, _WCH, _ACH)
    recv3d = jnp.concatenate([recv, _N + spread]).reshape(_NW, _WCH, _ACH)
    f = pl.kernel(
        _agg_body,
        out_type=[jax.ShapeDtypeStruct((_NC, _NP, _D), jnp.float32),
                  jax.ShapeDtypeStruct((_NC, _NP, _D), jnp.float32)],
        mesh=_mesh(),
        scratch_types=[
            pltpu.VMEM((_TNCH, _ACH), jnp.int32),
            pltpu.VMEM((_TNCH, _ACH), jnp.int32),
            pltpu.VMEM((_ACH, _D), jnp.float32),
            pltpu.VMEM((_ACH, _D), jnp.float32),
            pltpu.VMEM((_ACH, _D), jnp.float32),
            pltpu.VMEM_SHARED((_NP, _D), jnp.float32),
            pltpu.SemaphoreType.DMA,
            pltpu.SemaphoreType.DMA,
            pltpu.SemaphoreType.DMA,
            pltpu.SemaphoreType.DMA,
            pltpu.SemaphoreType.DMA,
        ],
    )
    return f(tab1, tab2, send3d, recv3d)


def _feat_body(x_ref, cx_ref, w_ref, b_ref, dsp_ref, h1_ref, h2_ref):
    deg = dsp_ref[0] + dsp_ref[1]
    scale = lax.rsqrt(jnp.maximum(deg, 1.0))[:, None]
    wm = w_ref[...]
    bv = b_ref[...]
    h1_ref[...] = (jnp.dot(x_ref[...], wm,
                           preferred_element_type=jnp.float32) + bv) * scale
    h2_ref[...] = (jnp.dot(cx_ref[...], wm,
                           preferred_element_type=jnp.float32) + bv) * scale


def _feat_call(x, c_x, W, b2, degs_p):
    return pl.pallas_call(
        _feat_body,
        grid=(_GRID,),
        in_specs=[
            pl.BlockSpec((_RB, _D), lambda i: (i, 0)),
            pl.BlockSpec((_RB, _D), lambda i: (i, 0)),
            pl.BlockSpec((_D, _D), lambda i: (0, 0)),
            pl.BlockSpec((1, _D), lambda i: (0, 0)),
            pl.BlockSpec((_NC, _RB), lambda i: (0, i)),
        ],
        out_specs=[
            pl.BlockSpec((_RB, _D), lambda i: (i, 0)),
            pl.BlockSpec((_RB, _D), lambda i: (i, 0)),
        ],
        out_shape=[jax.ShapeDtypeStruct((_N, _D), jnp.float32),
                   jax.ShapeDtypeStruct((_N, _D), jnp.float32)],
    )(x, c_x, W, b2, degs_p)


def _selu(v):
    return _SELU_SCALE * jnp.where(v > 0, v, _SELU_ALPHA * (jnp.exp(v) - 1.0))


def _node_body(a1_ref, a2_ref, drp_ref, n1_ref, n2_ref, emb_ref, cs_ref,
               cs_s):
    i = pl.program_id(0)
    agg1 = a1_ref[0] + a1_ref[1]
    agg2 = a2_ref[0] + a2_ref[1]
    deg = drp_ref[0] + drp_ref[1]
    invr = lax.rsqrt(jnp.maximum(deg, 1.0))[:, None]
    n1 = _selu(agg1 * invr)
    n2 = _selu(agg2 * invr)
    n1_ref[...] = n1.astype(jnp.bfloat16)
    n2_ref[...] = n2.astype(jnp.bfloat16)
    nrm = jnp.sqrt(jnp.sum(n1 * n1, axis=1, keepdims=True))
    emb_ref[...] = n1 / (nrm + 1e-12)
    vrow = (lax.broadcasted_iota(jnp.int32, (_RB, 1), 0) + i * _RB) < _N
    bs = jnp.sum(jnp.where(vrow, n1, 0.0), axis=0, keepdims=True)

    @pl.when(i == 0)
    def _():
        cs_s[...] = bs

    @pl.when(i > 0)
    def _():
        cs_s[...] = cs_s[...] + bs

    @pl.when(i == _GRID - 1)
    def _():
        cs_ref[...] = cs_s[...]


def _node_call(agg1_p, agg2_p, degr_p):
    return pl.pallas_call(
        _node_body,
        grid=(_GRID,),
        in_specs=[
            pl.BlockSpec((_NC, _RB, _D), lambda i: (0, i, 0)),
            pl.BlockSpec((_NC, _RB, _D), lambda i: (0, i, 0)),
            pl.BlockSpec((_NC, _RB), lambda i: (0, i)),
        ],
        out_specs=[
            pl.BlockSpec((_RB, _D), lambda i: (i, 0)),
            pl.BlockSpec((_RB, _D), lambda i: (i, 0)),
            pl.BlockSpec((_RB, _D), lambda i: (i, 0)),
            pl.BlockSpec((1, _D), lambda i: (0, 0)),
        ],
        out_shape=[jax.ShapeDtypeStruct((_N, _D), jnp.bfloat16),
                   jax.ShapeDtypeStruct((_N, _D), jnp.bfloat16),
                   jax.ShapeDtypeStruct((_N, _D), jnp.float32),
                   jax.ShapeDtypeStruct((1, _D), jnp.float32)],
        scratch_shapes=[pltpu.VMEM((1, _D), jnp.float32)],
    )(agg1_p, agg2_p, degr_p)


def _head_body(n1_ref, n2_ref, emb_ref, cen_ref, wbl_ref, cs_ref,
               l1_ref, l2_ref, rid_ref, loss_ref,
               rv_s, ri_s, ls_s):
    i = pl.program_id(0)
    cs = cs_ref[0]
    summary = jax.nn.sigmoid(cs * (1.0 / _N))
    v = jnp.sum(wbl_ref[...] * summary[None, :], axis=1)
    n1 = n1_ref[...].astype(jnp.float32)
    n2 = n2_ref[...].astype(jnp.float32)
    l1_ref[...] = jnp.dot(n1, v[:, None],
                          preferred_element_type=jnp.float32)[:, 0]
    l2_ref[...] = jnp.dot(n2, v[:, None],
                          preferred_element_type=jnp.float32)[:, 0]

    emb = emb_ref[...]
    cen = cen_ref[...]
    e2 = jnp.sum(emb * emb, axis=1)[:, None]
    c2 = jnp.sum(cen * cen, axis=1)[None, :]
    dots = lax.dot_general(emb, cen, (((1,), (1,)), ((), ())),
                           preferred_element_type=jnp.float32)
    sq = e2 + c2 - 2.0 * dots
    rowid = lax.broadcasted_iota(jnp.int32, (_RB, _K), 0)
    valid = (rowid + i * _RB) < _N
    dm = jnp.where(valid, sq, jnp.inf)
    bmin = jnp.min(dm, axis=0)
    barg = jnp.min(jnp.where(dm == bmin[None, :], rowid, jnp.int32(2 ** 30)),
                   axis=0) + i * _RB
    rmin = jnp.sqrt(jnp.maximum(jnp.min(dm, axis=1), 1e-12))
    bloss = jnp.sum(jnp.where(valid[:, 0], rmin, 0.0))

    @pl.when(i == 0)
    def _():
        rv_s[...] = bmin
        ri_s[...] = barg
        ls_s[0, 0] = bloss

    @pl.when(i > 0)
    def _():
        better = bmin < rv_s[...]
        rv_s[...] = jnp.where(better, bmin, rv_s[...])
        ri_s[...] = jnp.where(better, barg, ri_s[...])
        ls_s[0, 0] = ls_s[0, 0] + bloss

    @pl.when(i == _GRID - 1)
    def _():
        rid_ref[...] = ri_s[...]
        loss_ref[0, 0] = ls_s[0, 0]


def _head_call(n1, n2, emb_p, centers, w_bl, cs):
    return pl.pallas_call(
        _head_body,
        grid=(_GRID,),
        in_specs=[
            pl.BlockSpec((_RB, _D), lambda i: (i, 0)),
            pl.BlockSpec((_RB, _D), lambda i: (i, 0)),
            pl.BlockSpec((_RB, _D), lambda i: (i, 0)),
            pl.BlockSpec((_K, _D), lambda i: (0, 0)),
            pl.BlockSpec((_D, _D), lambda i: (0, 0)),
            pl.BlockSpec((1, _D), lambda i: (0, 0)),
        ],
        out_specs=[
            pl.BlockSpec((_RB,), lambda i: (i,)),
            pl.BlockSpec((_RB,), lambda i: (i,)),
            pl.BlockSpec((_K,), lambda i: (0,)),
            pl.BlockSpec(memory_space=pltpu.SMEM),
        ],
        out_shape=[jax.ShapeDtypeStruct((_N,), jnp.float32),
                   jax.ShapeDtypeStruct((_N,), jnp.float32),
                   jax.ShapeDtypeStruct((_K,), jnp.int32),
                   jax.ShapeDtypeStruct((1, 1), jnp.float32)],
        scratch_shapes=[pltpu.VMEM((_K,), jnp.float32),
                        pltpu.VMEM((_K,), jnp.int32),
                        pltpu.SMEM((1, 1), jnp.float32)],
    )(n1, n2, emb_p, centers, w_bl, cs)


def kernel(x, c_x, edge_index, W, b, w_bl, centers):
    edges_flat = edge_index.astype(jnp.int32).reshape(-1)
    send = edges_flat[:_E]
    recv = edges_flat[_E:]

    degs_p, degr_p = _deg_call(edges_flat)
    h1, h2 = _feat_call(x, c_x, W, b.reshape(1, _D), degs_p)
    agg1_p, agg2_p = _agg_call(h1, h2, send, recv)
    n1, n2, emb, cs = _node_call(agg1_p, agg2_p, degr_p)
    l1, l2, rep_ids, loss = _head_call(n1, n2, emb, centers, w_bl, cs)

    logits = jnp.concatenate([l1, l2])
    return (emb, centers, rep_ids, loss[0, 0], logits)


# speedup vs baseline: 1.2946x; 1.0276x over previous
import jax
import jax.numpy as jnp
from jax import lax
from jax.experimental import pallas as pl
from jax.experimental.pallas import tpu as pltpu
from jax.experimental.pallas import tpu_sc as plsc

_NC = 2
_NS = 16
_NW = _NC * _NS

_N = 10000
_NP = 10240
_E = 320000
_D = 128
_K = 512

_RB = 1024
_GRID = _NP // _RB

_DCH = 2560

_SELU_ALPHA = 1.6732632423543772
_SELU_SCALE = 1.0507009873554805


def _mesh():
    return plsc.VectorSubcoreMesh(
        core_axis_name="c", subcore_axis_name="s",
        num_cores=_NC, num_subcores=_NS)


_DNCH = _E // _DCH


def _deg_body(edges_hbm, outs_hbm, outr_hbm,
              idx_v, ones_v, zb_v, hs_sh, hr_sh):
    cid = lax.axis_index("c")
    sid = lax.axis_index("s")
    w = cid * _NS + sid

    def fill_ones(i, carry):
        ones_v[pl.ds(i * 16, 16)] = jnp.full((16,), 1.0, jnp.float32)
        return carry
    lax.fori_loop(0, _DCH // 16, fill_ones, 0)

    def fill_zero(i, carry):
        zb_v[pl.ds(i * 16, 16)] = jnp.zeros((16,), jnp.float32)
        return carry
    lax.fori_loop(0, 640 // 16, fill_zero, 0)

    pltpu.sync_copy(zb_v, hs_sh.at[pl.ds(sid * 640, 640)])
    pltpu.sync_copy(zb_v, hr_sh.at[pl.ds(sid * 640, 640)])
    plsc.subcore_barrier()

    nch = jnp.where(w + 3 * _NW < _DNCH, 4, 3)

    def step(j, carry):
        base = (w + j * _NW) * _DCH
        pltpu.sync_copy(edges_hbm.at[0, pl.ds(base, _DCH)], idx_v)
        pltpu.sync_copy(ones_v, hs_sh.at[idx_v], add=True)
        pltpu.sync_copy(edges_hbm.at[1, pl.ds(base, _DCH)], idx_v)
        pltpu.sync_copy(ones_v, hr_sh.at[idx_v], add=True)
        return carry
    lax.fori_loop(0, nch, step, 0)
    plsc.subcore_barrier()

    pltpu.sync_copy(hs_sh.at[pl.ds(sid * 640, 640)],
                    outs_hbm.at[cid, pl.ds(sid * 640, 640)])
    pltpu.sync_copy(hr_sh.at[pl.ds(sid * 640, 640)],
                    outr_hbm.at[cid, pl.ds(sid * 640, 640)])


def _deg_call(edge2d):
    f = pl.kernel(
        _deg_body,
        out_type=[jax.ShapeDtypeStruct((_NC, _NP), jnp.float32),
                  jax.ShapeDtypeStruct((_NC, _NP), jnp.float32)],
        mesh=_mesh(),
        scratch_types=[
            pltpu.VMEM((_DCH,), jnp.int32),
            pltpu.VMEM((_DCH,), jnp.float32),
            pltpu.VMEM((640,), jnp.float32),
            pltpu.VMEM_SHARED((_NP,), jnp.float32),
            pltpu.VMEM_SHARED((_NP,), jnp.float32),
        ],
    )
    return f(edge2d)


_ACH = 88
_WCH = 120
_TNCH = 40
_EPAD = _NW * _WCH * _ACH


def _agg_body(tab1_hbm, tab2_hbm, send3d_hbm, recv3d_hbm, out1_hbm, out2_hbm,
              sidx_v, ridx_v, buf0, buf1, buf2,
              acc_sh, gs0, gs1, gs2, ss0, ss1):
    cid = lax.axis_index("c")
    sid = lax.axis_index("s")
    w = cid * _NS + sid
    bufs = (buf0, buf1, buf2)
    gsems = (gs0, gs1, gs2)
    ssems = (ss0, ss1)

    nz = (640 + _ACH - 1) // _ACH

    def zero_own(zsrc):
        def zrow(i, carry):
            for cix in range(_D // 16):
                zsrc[i, pl.ds(cix * 16, 16)] = jnp.zeros((16,), jnp.float32)
            return carry
        lax.fori_loop(0, _ACH, zrow, 0)
        for j in range(nz):
            r0 = sid * 640 + j * _ACH
            rows = min(_ACH, 640 - j * _ACH)
            pltpu.sync_copy(zsrc.at[pl.ds(0, rows)],
                            acc_sh.at[pl.ds(r0, rows)])

    def copy_own(out_hbm):
        for j in range(nz):
            r0 = sid * 640 + j * _ACH
            rows = min(_ACH, 640 - j * _ACH)
            pltpu.sync_copy(acc_sh.at[pl.ds(r0, rows)],
                            out_hbm.at[cid, pl.ds(r0, rows)])

    for pi, (tab_hbm, out_hbm) in enumerate(
            ((tab1_hbm, out1_hbm), (tab2_hbm, out2_hbm))):
        def gstart(i, buf, sem):
            pltpu.make_async_copy(tab_hbm.at[sidx_v.at[i]], buf, sem).start()

        def gwait(i, buf, sem):
            pltpu.make_async_copy(tab_hbm.at[sidx_v.at[i]], buf, sem).wait()

        def sstart(i, buf, sem):
            pltpu.make_async_copy(buf, acc_sh.at[ridx_v.at[i]],
                                  sem).start(add=True)

        def swait(i, buf, sem):
            pltpu.make_async_copy(buf, acc_sh.at[ridx_v.at[i]], sem).wait()

        for t in range(_WCH // _TNCH):
            pltpu.sync_copy(send3d_hbm.at[w, pl.ds(t * _TNCH, _TNCH)], sidx_v)
            pltpu.sync_copy(recv3d_hbm.at[w, pl.ds(t * _TNCH, _TNCH)], ridx_v)

            gstart(0, buf0, gs0)
            gstart(1, buf1, gs1)

            if t == 0:
                if pi == 1:
                    copy_own(out1_hbm)
                zero_own(buf2)
                plsc.subcore_barrier()

            def step(i, carry):
                for k in range(6):
                    @pl.when(i % 6 == k)
                    def _(k=k):
                        b = k % 3
                        s = k % 2
                        gwait(i, bufs[b], gsems[b])
                        sstart(i, bufs[b], ssems[s])

                        @pl.when(i == 0)
                        def _():
                            gstart(i + 2, bufs[(b + 2) % 3],
                                   gsems[(b + 2) % 3])

                        @pl.when((i >= 1) & (i + 2 < _TNCH))
                        def _():
                            swait(i - 1, bufs[(b + 2) % 3], ssems[1 - s])
                            gstart(i + 2, bufs[(b + 2) % 3],
                                   gsems[(b + 2) % 3])
                return carry
            lax.fori_loop(0, _TNCH, step, 0)
            swait(_TNCH - 3, bufs[(_TNCH - 3) % 3], ssems[(_TNCH - 3) % 2])
            swait(_TNCH - 2, bufs[(_TNCH - 2) % 3], ssems[(_TNCH - 2) % 2])
            swait(_TNCH - 1, bufs[(_TNCH - 1) % 3], ssems[(_TNCH - 1) % 2])
        plsc.subcore_barrier()

    copy_own(out2_hbm)


def _agg_call(tab1, tab2, send, recv):
    pad = _EPAD - _E
    spread = jnp.arange(pad, dtype=jnp.int32) % (_NP - _N)
    send3d = jnp.concatenate([send, spread]).reshape(_NW, _WCH, _ACH)
    recv3d = jnp.concatenate([recv, _N + spread]).reshape(_NW, _WCH, _ACH)
    f = pl.kernel(
        _agg_body,
        out_type=[jax.ShapeDtypeStruct((_NC, _NP, _D), jnp.float32),
                  jax.ShapeDtypeStruct((_NC, _NP, _D), jnp.float32)],
        mesh=_mesh(),
        scratch_types=[
            pltpu.VMEM((_TNCH, _ACH), jnp.int32),
            pltpu.VMEM((_TNCH, _ACH), jnp.int32),
            pltpu.VMEM((_ACH, _D), jnp.float32),
            pltpu.VMEM((_ACH, _D), jnp.float32),
            pltpu.VMEM((_ACH, _D), jnp.float32),
            pltpu.VMEM_SHARED((_NP, _D), jnp.float32),
            pltpu.SemaphoreType.DMA,
            pltpu.SemaphoreType.DMA,
            pltpu.SemaphoreType.DMA,
            pltpu.SemaphoreType.DMA,
            pltpu.SemaphoreType.DMA,
        ],
    )
    return f(tab1, tab2, send3d, recv3d)


def _feat_body(x_ref, cx_ref, w_ref, b_ref, dsp_ref, h1_ref, h2_ref):
    deg = dsp_ref[0] + dsp_ref[1]
    scale = lax.rsqrt(jnp.maximum(deg, 1.0))[:, None]
    wm = w_ref[...]
    bv = b_ref[...]
    h1_ref[...] = (jnp.dot(x_ref[...], wm,
                           preferred_element_type=jnp.float32) + bv) * scale
    h2_ref[...] = (jnp.dot(cx_ref[...], wm,
                           preferred_element_type=jnp.float32) + bv) * scale


def _feat_call(x, c_x, W, b2, degs_p):
    return pl.pallas_call(
        _feat_body,
        grid=(_GRID,),
        in_specs=[
            pl.BlockSpec((_RB, _D), lambda i: (i, 0)),
            pl.BlockSpec((_RB, _D), lambda i: (i, 0)),
            pl.BlockSpec((_D, _D), lambda i: (0, 0)),
            pl.BlockSpec((1, _D), lambda i: (0, 0)),
            pl.BlockSpec((_NC, _RB), lambda i: (0, i)),
        ],
        out_specs=[
            pl.BlockSpec((_RB, _D), lambda i: (i, 0)),
            pl.BlockSpec((_RB, _D), lambda i: (i, 0)),
        ],
        out_shape=[jax.ShapeDtypeStruct((_N, _D), jnp.float32),
                   jax.ShapeDtypeStruct((_N, _D), jnp.float32)],
    )(x, c_x, W, b2, degs_p)


def _selu(v):
    return _SELU_SCALE * jnp.where(v > 0, v, _SELU_ALPHA * (jnp.exp(v) - 1.0))


def _node_body(a1_ref, a2_ref, drp_ref, n1_ref, n2_ref, emb_ref, cs_ref,
               cs_s):
    i = pl.program_id(0)
    agg1 = a1_ref[0] + a1_ref[1]
    agg2 = a2_ref[0] + a2_ref[1]
    deg = drp_ref[0] + drp_ref[1]
    invr = lax.rsqrt(jnp.maximum(deg, 1.0))[:, None]
    n1 = _selu(agg1 * invr)
    n2 = _selu(agg2 * invr)
    n1_ref[...] = n1.astype(jnp.bfloat16)
    n2_ref[...] = n2.astype(jnp.bfloat16)
    nrm = jnp.sqrt(jnp.sum(n1 * n1, axis=1, keepdims=True))
    emb_ref[...] = n1 / (nrm + 1e-12)
    vrow = (lax.broadcasted_iota(jnp.int32, (_RB, 1), 0) + i * _RB) < _N
    bs = jnp.sum(jnp.where(vrow, n1, 0.0), axis=0, keepdims=True)

    @pl.when(i == 0)
    def _():
        cs_s[...] = bs

    @pl.when(i > 0)
    def _():
        cs_s[...] = cs_s[...] + bs

    @pl.when(i == _GRID - 1)
    def _():
        cs_ref[...] = cs_s[...]


def _node_call(agg1_p, agg2_p, degr_p):
    return pl.pallas_call(
        _node_body,
        grid=(_GRID,),
        in_specs=[
            pl.BlockSpec((_NC, _RB, _D), lambda i: (0, i, 0)),
            pl.BlockSpec((_NC, _RB, _D), lambda i: (0, i, 0)),
            pl.BlockSpec((_NC, _RB), lambda i: (0, i)),
        ],
        out_specs=[
            pl.BlockSpec((_RB, _D), lambda i: (i, 0)),
            pl.BlockSpec((_RB, _D), lambda i: (i, 0)),
            pl.BlockSpec((_RB, _D), lambda i: (i, 0)),
            pl.BlockSpec((1, _D), lambda i: (0, 0)),
        ],
        out_shape=[jax.ShapeDtypeStruct((_N, _D), jnp.bfloat16),
                   jax.ShapeDtypeStruct((_N, _D), jnp.bfloat16),
                   jax.ShapeDtypeStruct((_N, _D), jnp.float32),
                   jax.ShapeDtypeStruct((1, _D), jnp.float32)],
        scratch_shapes=[pltpu.VMEM((1, _D), jnp.float32)],
    )(agg1_p, agg2_p, degr_p)


def _head_body(n1_ref, n2_ref, emb_ref, cen_ref, wbl_ref, cs_ref,
               l1_ref, l2_ref, rid_ref, loss_ref,
               rv_s, ri_s, ls_s):
    i = pl.program_id(0)
    cs = cs_ref[0]
    summary = jax.nn.sigmoid(cs * (1.0 / _N))
    v = jnp.sum(wbl_ref[...] * summary[None, :], axis=1)
    n1 = n1_ref[...].astype(jnp.float32)
    n2 = n2_ref[...].astype(jnp.float32)
    l1_ref[...] = jnp.dot(n1, v[:, None],
                          preferred_element_type=jnp.float32)[:, 0]
    l2_ref[...] = jnp.dot(n2, v[:, None],
                          preferred_element_type=jnp.float32)[:, 0]

    emb = emb_ref[...]
    cen = cen_ref[...]
    e2 = jnp.sum(emb * emb, axis=1)[:, None]
    c2 = jnp.sum(cen * cen, axis=1)[None, :]
    dots = lax.dot_general(emb, cen, (((1,), (1,)), ((), ())),
                           preferred_element_type=jnp.float32)
    sq = e2 + c2 - 2.0 * dots
    rowid = lax.broadcasted_iota(jnp.int32, (_RB, _K), 0)
    nvalid = _N - i * _RB
    dm = jnp.where(rowid < nvalid, sq, jnp.inf)
    bmin = jnp.min(dm, axis=0)
    barg = jnp.min(jnp.where(dm == bmin[None, :], rowid, jnp.int32(2 ** 30)),
                   axis=0) + i * _RB
    rmin = jnp.sqrt(jnp.maximum(jnp.min(dm, axis=1), 1e-12))
    bloss = jnp.sum(jnp.where(rowid[:, 0] < nvalid, rmin, 0.0))

    @pl.when(i == 0)
    def _():
        rv_s[...] = bmin
        ri_s[...] = barg
        ls_s[0, 0] = bloss

    @pl.when(i > 0)
    def _():
        better = bmin < rv_s[...]
        rv_s[...] = jnp.where(better, bmin, rv_s[...])
        ri_s[...] = jnp.where(better, barg, ri_s[...])
        ls_s[0, 0] = ls_s[0, 0] + bloss

    @pl.when(i == _GRID - 1)
    def _():
        rid_ref[...] = ri_s[...]
        loss_ref[0, 0] = ls_s[0, 0]


def _head_call(n1, n2, emb_p, centers, w_bl, cs):
    return pl.pallas_call(
        _head_body,
        grid=(_GRID,),
        in_specs=[
            pl.BlockSpec((_RB, _D), lambda i: (i, 0)),
            pl.BlockSpec((_RB, _D), lambda i: (i, 0)),
            pl.BlockSpec((_RB, _D), lambda i: (i, 0)),
            pl.BlockSpec((_K, _D), lambda i: (0, 0)),
            pl.BlockSpec((_D, _D), lambda i: (0, 0)),
            pl.BlockSpec((1, _D), lambda i: (0, 0)),
        ],
        out_specs=[
            pl.BlockSpec((_RB,), lambda i: (i,)),
            pl.BlockSpec((_RB,), lambda i: (i,)),
            pl.BlockSpec((_K,), lambda i: (0,)),
            pl.BlockSpec(memory_space=pltpu.SMEM),
        ],
        out_shape=[jax.ShapeDtypeStruct((_N,), jnp.float32),
                   jax.ShapeDtypeStruct((_N,), jnp.float32),
                   jax.ShapeDtypeStruct((_K,), jnp.int32),
                   jax.ShapeDtypeStruct((1, 1), jnp.float32)],
        scratch_shapes=[pltpu.VMEM((_K,), jnp.float32),
                        pltpu.VMEM((_K,), jnp.int32),
                        pltpu.SMEM((1, 1), jnp.float32)],
    )(n1, n2, emb_p, centers, w_bl, cs)


def kernel(x, c_x, edge_index, W, b, w_bl, centers):
    edge2d = edge_index.astype(jnp.int32)
    send = edge2d[0]
    recv = edge2d[1]

    degs_p, degr_p = _deg_call(edge2d)
    h1, h2 = _feat_call(x, c_x, W, b.reshape(1, _D), degs_p)
    agg1_p, agg2_p = _agg_call(h1, h2, send, recv)
    n1, n2, emb, cs = _node_call(agg1_p, agg2_p, degr_p)
    l1, l2, rep_ids, loss = _head_call(n1, n2, emb, centers, w_bl, cs)

    logits = jnp.concatenate([l1, l2])
    return (emb, centers, rep_ids, loss[0, 0], logits)
